# baseline (speedup 1.0000x reference)
import functools
import jax
import jax.numpy as jnp
from jax import lax
from jax.experimental import pallas as pl
from jax.experimental.pallas import tpu as pltpu
from jax.experimental.pallas import tpu_sc as plsc

SIZE = 4096
DIM = 512
B = 4
N = 4096
D_EMB = 512

NC = 2
NS = 16
NW = NC * NS
RW = N // NW
CE = 64

_mesh = plsc.VectorSubcoreMesh(core_axis_name="c", subcore_axis_name="s")


@functools.partial(
    pl.kernel,
    out_type=jax.ShapeDtypeStruct((B, N, D_EMB + DIM), jnp.float32),
    mesh=_mesh,
    scratch_types=[
        pltpu.VMEM((RW, DIM), jnp.float32),
        pltpu.VMEM((CE, D_EMB), jnp.float32),
    ],
)
def _sc_pos_embed(emb_hbm, pe_hbm, out_hbm, pe_v, emb_v):
    wid = lax.axis_index("s") * NC + lax.axis_index("c")
    base = wid * RW
    pltpu.sync_copy(pe_hbm.at[pl.ds(base, RW)], pe_v)
    for b in range(B):
        pltpu.sync_copy(pe_v, out_hbm.at[b, pl.ds(base, RW), pl.ds(D_EMB, DIM)])
    for b in range(B):
        for c in range(RW // CE):
            o = base + c * CE
            pltpu.sync_copy(emb_hbm.at[b, pl.ds(o, CE)], emb_v)
            pltpu.sync_copy(emb_v, out_hbm.at[b, pl.ds(o, CE), pl.ds(0, D_EMB)])


def kernel(emb, pe):
    pe_rows = jax.lax.slice(pe, (1, 0), (1 + N, DIM))
    return _sc_pos_embed(emb, pe_rows)


# speedup vs baseline: 1.0155x; 1.0155x over previous
import functools
import jax
import jax.numpy as jnp
from jax import lax
from jax.experimental import pallas as pl
from jax.experimental.pallas import tpu as pltpu
from jax.experimental.pallas import tpu_sc as plsc

SIZE = 4096
DIM = 512
B = 4
N = 4096
D_EMB = 512

NC = 2
NS = 16
NW = NC * NS
RW = N // NW
CE = 32

_mesh = plsc.VectorSubcoreMesh(core_axis_name="c", subcore_axis_name="s")


@functools.partial(
    pl.kernel,
    out_type=jax.ShapeDtypeStruct((B, N, D_EMB + DIM), jnp.float32),
    mesh=_mesh,
    scratch_types=[
        pltpu.VMEM((RW, DIM), jnp.float32),
        pltpu.VMEM((2, CE, D_EMB), jnp.float32),
        pltpu.SemaphoreType.DMA,
        pltpu.SemaphoreType.DMA,
        pltpu.SemaphoreType.DMA,
        pltpu.SemaphoreType.DMA,
    ],
)
def _sc_pos_embed(emb_hbm, pe_hbm, out_hbm, pe_v, emb_v, sem_pe, sem_g,
                  sem_s0, sem_s1):
    wid = lax.axis_index("s") * NC + lax.axis_index("c")
    base = wid * RW
    pltpu.sync_copy(pe_hbm.at[pl.ds(base, RW)], pe_v)
    pe_handles = [
        pltpu.async_copy(
            pe_v, out_hbm.at[b, pl.ds(base, RW), pl.ds(D_EMB, DIM)], sem_pe)
        for b in range(B)
    ]
    sem_s = (sem_s0, sem_s1)
    n_chunks = B * (RW // CE)
    s_handles = [None] * n_chunks
    for i in range(n_chunks):
        b, c = divmod(i, RW // CE)
        o = base + c * CE
        buf = i % 2
        if i >= 2:
            s_handles[i - 2].wait()
        pltpu.async_copy(emb_hbm.at[b, pl.ds(o, CE)], emb_v.at[buf],
                         sem_g).wait()
        s_handles[i] = pltpu.async_copy(
            emb_v.at[buf], out_hbm.at[b, pl.ds(o, CE), pl.ds(0, D_EMB)],
            sem_s[buf])
    s_handles[n_chunks - 2].wait()
    s_handles[n_chunks - 1].wait()
    for h in pe_handles:
        h.wait()


def kernel(emb, pe):
    pe_rows = jax.lax.slice(pe, (1, 0), (1 + N, DIM))
    return _sc_pos_embed(emb, pe_rows)
